# baseline (speedup 1.0000x reference)
import functools

import jax
import jax.numpy as jnp
from jax import lax
from jax.experimental import pallas as pl
from jax.experimental.pallas import tpu as pltpu
from jax.experimental.pallas import tpu_sc as plsc

VOCAB = 1000
D = 16
NC = 2
NS = 16
L = 16
NW = NC * NS
BT = 128
G_BT = 8
CB = BT * G_BT
OUTC = CB * 8
UNROLL = 8


@functools.lru_cache(maxsize=None)
def _build(b: int, h: int):
    n_dt = D // 8
    nbg = b // CB
    chunks = h * nbg
    assert b % CB == 0 and chunks % (2 * NW) == 0
    n_chunks = chunks // NW
    n_pairs = n_chunks // 2
    groups = CB // L
    hs = b * D
    dts = hs // n_dt

    mesh = plsc.VectorSubcoreMesh(core_axis_name="c", subcore_axis_name="s")

    @functools.partial(
        pl.kernel,
        out_type=jax.ShapeDtypeStruct((b * h * D,), jnp.float32),
        mesh=mesh,
        compiler_params=pltpu.CompilerParams(
            needs_layout_passes=False, disable_bounds_checks=True),
        scratch_types=[
            pltpu.VMEM((VOCAB * D,), jnp.float32),
            pltpu.VMEM((CB,), jnp.int32),
            pltpu.VMEM((CB,), jnp.int32),
            pltpu.VMEM((n_dt * OUTC,), jnp.float32),
            pltpu.VMEM((n_dt * OUTC,), jnp.float32),
            pltpu.SemaphoreType.DMA,
            pltpu.SemaphoreType.DMA,
            pltpu.SemaphoreType.DMA,
            pltpu.SemaphoreType.DMA,
            pltpu.SemaphoreType.DMA,
            pltpu.SemaphoreType.DMA,
        ],
    )
    def body(idx_hbm, tab_hbm, out_hbm, tab_v, idx0, idx1, ob0, ob1,
             isem0, isem1, osem00, osem01, osem10, osem11):
        idx_b = (idx0, idx1)
        ob = (ob0, ob1)
        isem = (isem0, isem1)
        osem = ((osem00, osem01), (osem10, osem11))

        wid = lax.axis_index("s") * NC + lax.axis_index("c")
        c0 = wid * n_chunks
        pltpu.sync_copy(tab_hbm, tab_v)
        iota = lax.iota(jnp.int32, L)
        rots = [(iota + d) & (D - 1) for d in range(D)]
        svecs = [(r >> 3) * OUTC + (r & 7) * BT + iota for r in rots]

        def idx_dma(c, buf):
            cg = c0 + c
            hh = cg // nbg
            btg = cg - hh * nbg
            src = pl.multiple_of(hh * b + btg * CB, CB)
            return pltpu.make_async_copy(
                idx_hbm.at[pl.ds(src, CB)], idx_b[buf], isem[buf])

        def out_dma(c, buf, dt):
            cg = c0 + c
            hh = cg // nbg
            btg = cg - hh * nbg
            dst = pl.multiple_of(hh * hs + dt * dts + btg * OUTC, OUTC)
            return pltpu.make_async_copy(
                ob[buf].at[pl.ds(dt * OUTC, OUTC)],
                out_hbm.at[pl.ds(dst, OUTC)], osem[buf][dt])

        idx_dma(0, 0).start()
        idx_dma(1, 1).start()

        def pair(p, carry):
            for buf in range(2):
                c = 2 * p + buf
                @pl.when(p > 0)
                def _():
                    out_dma(c - 2, buf, 0).wait()
                    out_dma(c - 2, buf, 1).wait()
                idx_dma(c, buf).wait()

                idx_ref = idx_b[buf]
                ob_ref = ob[buf]

                @plsc.parallel_loop(0, groups, 1, unroll=UNROLL)
                def _(g):
                    idx = idx_ref[pl.ds(g * L, L)]
                    gaddr = idx * D
                    btl = g // G_BT
                    goff = btl * (BT * 8) + (g - btl * G_BT) * L
                    for d in range(D):
                        diag = plsc.load_gather(tab_v, [gaddr + rots[d]])
                        plsc.store_scatter(ob_ref, [svecs[d] + goff], diag)

                out_dma(c, buf, 0).start()
                out_dma(c, buf, 1).start()
                @pl.when(c + 2 < n_chunks)
                def _():
                    idx_dma(c + 2, buf).start()
            return carry

        lax.fori_loop(0, n_pairs, pair, 0)
        out_dma(n_chunks - 2, 0, 0).wait()
        out_dma(n_chunks - 2, 0, 1).wait()
        out_dma(n_chunks - 1, 1, 0).wait()
        out_dma(n_chunks - 1, 1, 1).wait()

    return body


def kernel(inputs, table):
    b, h = inputs.shape
    idx_flat = inputs.T.reshape(-1).astype(jnp.int32)
    tab_flat = table.reshape(-1)
    out = _build(b, h)(idx_flat, tab_flat)
    out5 = out.reshape(h, D // 8, b // BT, 8, BT)
    return out5.transpose((2, 4, 0, 1, 3)).reshape(b, h, D)


# speedup vs baseline: 1.5813x; 1.5813x over previous
import functools

import jax
import jax.numpy as jnp
from jax import lax
from jax.experimental import pallas as pl
from jax.experimental.pallas import tpu as pltpu
from jax.experimental.pallas import tpu_sc as plsc

VOCAB = 1000
D = 16
NC = 2
NS = 16
L = 16
NW = NC * NS
BT = 128
G_BT = 16
CB = BT * G_BT
OUTC = CB * 8
UNROLL = 4


@functools.lru_cache(maxsize=None)
def _build(b: int, h: int):
    n_dt = D // 8
    nbg = b // CB
    chunks = h * nbg
    assert b % CB == 0 and chunks % (2 * NW) == 0
    n_chunks = chunks // NW
    n_pairs = n_chunks // 2
    groups = CB // L
    hs = b * D
    dts = hs // n_dt

    mesh = plsc.VectorSubcoreMesh(core_axis_name="c", subcore_axis_name="s")

    @functools.partial(
        pl.kernel,
        out_type=jax.ShapeDtypeStruct((b * h * D,), jnp.float32),
        mesh=mesh,
        compiler_params=pltpu.CompilerParams(
            needs_layout_passes=False, disable_bounds_checks=True),
        scratch_types=[
            pltpu.VMEM((VOCAB * D,), jnp.float32),
            pltpu.VMEM((CB,), jnp.int32),
            pltpu.VMEM((CB,), jnp.int32),
            pltpu.VMEM((n_dt * OUTC,), jnp.float32),
            pltpu.VMEM((n_dt * OUTC,), jnp.float32),
            pltpu.SemaphoreType.DMA,
            pltpu.SemaphoreType.DMA,
            pltpu.SemaphoreType.DMA,
            pltpu.SemaphoreType.DMA,
            pltpu.SemaphoreType.DMA,
            pltpu.SemaphoreType.DMA,
        ],
    )
    def body(idx_hbm, tab_hbm, out_hbm, tab_v, idx0, idx1, ob0, ob1,
             isem0, isem1, osem00, osem01, osem10, osem11):
        idx_b = (idx0, idx1)
        ob = (ob0, ob1)
        isem = (isem0, isem1)
        osem = ((osem00, osem01), (osem10, osem11))

        wid = lax.axis_index("s") * NC + lax.axis_index("c")
        c0 = wid * n_chunks
        pltpu.sync_copy(tab_hbm, tab_v)
        iota = lax.iota(jnp.int32, L)
        rots = [(iota + d) & (D - 1) for d in range(D)]
        svecs = [(r >> 3) * OUTC + (r & 7) * BT + iota for r in rots]

        def idx_dma(c, buf):
            cg = c0 + c
            hh = cg // nbg
            btg = cg - hh * nbg
            src = pl.multiple_of(hh * b + btg * CB, CB)
            return pltpu.make_async_copy(
                idx_hbm.at[pl.ds(src, CB)], idx_b[buf], isem[buf])

        def out_dma(c, buf, dt):
            cg = c0 + c
            hh = cg // nbg
            btg = cg - hh * nbg
            dst = pl.multiple_of(hh * hs + dt * dts + btg * OUTC, OUTC)
            return pltpu.make_async_copy(
                ob[buf].at[pl.ds(dt * OUTC, OUTC)],
                out_hbm.at[pl.ds(dst, OUTC)], osem[buf][dt])

        idx_dma(0, 0).start()
        idx_dma(1, 1).start()

        def pair(p, carry):
            for buf in range(2):
                c = 2 * p + buf
                @pl.when(p > 0)
                def _():
                    out_dma(c - 2, buf, 0).wait()
                    out_dma(c - 2, buf, 1).wait()
                idx_dma(c, buf).wait()

                idx_ref = idx_b[buf]
                ob_ref = ob[buf]

                @plsc.parallel_loop(0, groups, 1, unroll=UNROLL)
                def _(g):
                    idx = idx_ref[pl.ds(g * L, L)]
                    gaddr = idx * D
                    btl = g // (BT // L)
                    goff = btl * (BT * 8) + (g - btl * (BT // L)) * L
                    for d in range(D):
                        diag = plsc.load_gather(tab_v, [gaddr + rots[d]])
                        plsc.store_scatter(ob_ref, [svecs[d] + goff], diag)

                out_dma(c, buf, 0).start()
                out_dma(c, buf, 1).start()
                @pl.when(c + 2 < n_chunks)
                def _():
                    idx_dma(c + 2, buf).start()
            return carry

        lax.fori_loop(0, n_pairs, pair, 0)
        out_dma(n_chunks - 2, 0, 0).wait()
        out_dma(n_chunks - 2, 0, 1).wait()
        out_dma(n_chunks - 1, 1, 0).wait()
        out_dma(n_chunks - 1, 1, 1).wait()

    return body


def kernel(inputs, table):
    b, h = inputs.shape
    idx_flat = inputs.T.reshape(-1).astype(jnp.int32)
    tab_flat = table.reshape(-1)
    out = _build(b, h)(idx_flat, tab_flat)
    out5 = out.reshape(h, D // 8, b // BT, 8, BT)
    return out5.transpose((2, 4, 0, 1, 3)).reshape(b, h, D)


# speedup vs baseline: 1.5889x; 1.0049x over previous
import functools

import jax
import jax.numpy as jnp
from jax import lax
from jax.experimental import pallas as pl
from jax.experimental.pallas import tpu as pltpu
from jax.experimental.pallas import tpu_sc as plsc

VOCAB = 1000
D = 16
NC = 2
NS = 16
L = 16
NW = NC * NS
BT = 128
G_BT = 16
CB = BT * G_BT
OUTC = CB * 8
UNROLL = 2


@functools.lru_cache(maxsize=None)
def _build(b: int, h: int):
    n_dt = D // 8
    nbg = b // CB
    chunks = h * nbg
    assert b % CB == 0 and chunks % (2 * NW) == 0
    n_chunks = chunks // NW
    n_pairs = n_chunks // 2
    groups = CB // L
    hs = b * D
    dts = hs // n_dt

    mesh = plsc.VectorSubcoreMesh(core_axis_name="c", subcore_axis_name="s")

    @functools.partial(
        pl.kernel,
        out_type=jax.ShapeDtypeStruct((b * h * D,), jnp.float32),
        mesh=mesh,
        compiler_params=pltpu.CompilerParams(
            needs_layout_passes=False, disable_bounds_checks=True),
        scratch_types=[
            pltpu.VMEM((VOCAB * D,), jnp.float32),
            pltpu.VMEM((CB,), jnp.int32),
            pltpu.VMEM((CB,), jnp.int32),
            pltpu.VMEM((n_dt * OUTC,), jnp.float32),
            pltpu.VMEM((n_dt * OUTC,), jnp.float32),
            pltpu.SemaphoreType.DMA,
            pltpu.SemaphoreType.DMA,
            pltpu.SemaphoreType.DMA,
            pltpu.SemaphoreType.DMA,
            pltpu.SemaphoreType.DMA,
            pltpu.SemaphoreType.DMA,
        ],
    )
    def body(idx_hbm, tab_hbm, out_hbm, tab_v, idx0, idx1, ob0, ob1,
             isem0, isem1, osem00, osem01, osem10, osem11):
        idx_b = (idx0, idx1)
        ob = (ob0, ob1)
        isem = (isem0, isem1)
        osem = ((osem00, osem01), (osem10, osem11))

        wid = lax.axis_index("s") * NC + lax.axis_index("c")
        c0 = wid * n_chunks
        pltpu.sync_copy(tab_hbm, tab_v)
        iota = lax.iota(jnp.int32, L)
        rots = [(iota + d) & (D - 1) for d in range(D)]
        svecs = [(r >> 3) * OUTC + (r & 7) * BT + iota for r in rots]

        def idx_dma(c, buf):
            cg = c0 + c
            hh = cg // nbg
            btg = cg - hh * nbg
            src = pl.multiple_of(hh * b + btg * CB, CB)
            return pltpu.make_async_copy(
                idx_hbm.at[pl.ds(src, CB)], idx_b[buf], isem[buf])

        def out_dma(c, buf, dt):
            cg = c0 + c
            hh = cg // nbg
            btg = cg - hh * nbg
            dst = pl.multiple_of(hh * hs + dt * dts + btg * OUTC, OUTC)
            return pltpu.make_async_copy(
                ob[buf].at[pl.ds(dt * OUTC, OUTC)],
                out_hbm.at[pl.ds(dst, OUTC)], osem[buf][dt])

        idx_dma(0, 0).start()
        idx_dma(1, 1).start()

        def pair(p, carry):
            for buf in range(2):
                c = 2 * p + buf
                @pl.when(p > 0)
                def _():
                    out_dma(c - 2, buf, 0).wait()
                    out_dma(c - 2, buf, 1).wait()
                idx_dma(c, buf).wait()

                idx_ref = idx_b[buf]
                ob_ref = ob[buf]

                @plsc.parallel_loop(0, groups, 1, unroll=UNROLL)
                def _(g):
                    idx = idx_ref[pl.ds(g * L, L)]
                    gaddr = idx * D
                    btl = g // (BT // L)
                    goff = btl * (BT * 8) + (g - btl * (BT // L)) * L
                    for d in range(D):
                        diag = plsc.load_gather(tab_v, [gaddr + rots[d]])
                        plsc.store_scatter(ob_ref, [svecs[d] + goff], diag)

                out_dma(c, buf, 0).start()
                out_dma(c, buf, 1).start()
                @pl.when(c + 2 < n_chunks)
                def _():
                    idx_dma(c + 2, buf).start()
            return carry

        lax.fori_loop(0, n_pairs, pair, 0)
        out_dma(n_chunks - 2, 0, 0).wait()
        out_dma(n_chunks - 2, 0, 1).wait()
        out_dma(n_chunks - 1, 1, 0).wait()
        out_dma(n_chunks - 1, 1, 1).wait()

    return body


def kernel(inputs, table):
    b, h = inputs.shape
    idx_flat = inputs.T.reshape(-1).astype(jnp.int32)
    tab_flat = table.reshape(-1)
    out = _build(b, h)(idx_flat, tab_flat)
    out5 = out.reshape(h, D // 8, b // BT, 8, BT)
    return out5.transpose((2, 4, 0, 1, 3)).reshape(b, h, D)
